# Initial kernel scaffold; baseline (speedup 1.0000x reference)
import jax
import jax.numpy as jnp
from jax.experimental import pallas as pl


def kernel(hidden_states, Wg, W1, W3, W2):
    raise NotImplementedError("write your pallas kernel here")


# speedup vs baseline: 1.2368x; 1.2368x over previous
import functools

import jax
import jax.numpy as jnp
from jax.experimental import pallas as pl
from jax.experimental.pallas import tpu as pltpu

B, S, D = 1, 2048, 768
E, F, K = 64, 384, 8
T = B * S


def _router_kernel(x_ref, wg_ref, combine_ref):
    logits = jax.lax.dot_general(
        x_ref[...], wg_ref[...], (((1,), (1,)), ((), ())),
        preferred_element_type=jnp.float32)
    m = jnp.max(logits, axis=-1, keepdims=True)
    ex = jnp.exp(logits - m)
    probs = ex / jnp.sum(ex, axis=-1, keepdims=True)

    lane = jax.lax.broadcasted_iota(jnp.int32, probs.shape, 1)
    cur = probs
    mask = jnp.zeros(probs.shape, dtype=jnp.bool_)
    for _ in range(K):
        mx = jnp.max(cur, axis=-1, keepdims=True)
        is_max = cur == mx
        first = jnp.min(jnp.where(is_max, lane, E), axis=-1, keepdims=True)
        sel = lane == first
        mask = mask | sel
        cur = jnp.where(sel, -jnp.inf, cur)

    picked = jnp.where(mask, probs, 0.0)
    combine_ref[...] = picked / jnp.sum(picked, axis=-1, keepdims=True)


def _expert_kernel(x_ref, w1_ref, w3_ref, w2_ref, combine_ref, out_ref):
    e = pl.program_id(0)

    h1 = jax.lax.dot_general(
        x_ref[...], w1_ref[0], (((1,), (0,)), ((), ())),
        preferred_element_type=jnp.float32)
    h3 = jax.lax.dot_general(
        x_ref[...], w3_ref[0], (((1,), (0,)), ((), ())),
        preferred_element_type=jnp.float32)
    h = (h1 * jax.lax.logistic(h1) * h3).astype(jnp.bfloat16)
    y = jax.lax.dot_general(
        h, w2_ref[0], (((1,), (0,)), ((), ())),
        preferred_element_type=jnp.float32)
    lane = jax.lax.broadcasted_iota(jnp.int32, (T, E), 1)
    w = jnp.sum(jnp.where(lane == e, combine_ref[...], 0.0),
                axis=1, keepdims=True)
    contrib = w * y

    @pl.when(e == 0)
    def _():
        out_ref[...] = contrib

    @pl.when(e != 0)
    def _():
        out_ref[...] += contrib


@functools.partial(jax.jit, static_argnames=())
def kernel(hidden_states, Wg, W1, W3, W2):
    old_shape = hidden_states.shape
    x = hidden_states.reshape(-1, old_shape[-1])

    combine = pl.pallas_call(
        _router_kernel,
        out_shape=jax.ShapeDtypeStruct((T, E), jnp.float32),
    )(x, Wg)

    xb = x.astype(jnp.bfloat16)
    W1b = W1.astype(jnp.bfloat16)
    W3b = W3.astype(jnp.bfloat16)
    W2b = W2.astype(jnp.bfloat16)

    out = pl.pallas_call(
        _expert_kernel,
        grid=(E,),
        in_specs=[
            pl.BlockSpec((T, D), lambda e: (0, 0)),
            pl.BlockSpec((1, D, F), lambda e: (e, 0, 0)),
            pl.BlockSpec((1, D, F), lambda e: (e, 0, 0)),
            pl.BlockSpec((1, F, D), lambda e: (e, 0, 0)),
            pl.BlockSpec((T, E), lambda e: (0, 0)),
        ],
        out_specs=pl.BlockSpec((T, D), lambda e: (0, 0)),
        out_shape=jax.ShapeDtypeStruct((T, D), jnp.float32),
        compiler_params=pltpu.CompilerParams(
            dimension_semantics=("arbitrary",),
        ),
    )(xb, W1b, W3b, W2b, combine)

    return out.reshape(old_shape)
